# baseline (speedup 1.0000x reference)
import functools

import jax
import jax.numpy as jnp
from jax import lax
from jax.experimental import pallas as pl
from jax.experimental.pallas import tpu as pltpu
from jax.experimental.pallas import tpu_sc as plsc


def _mm_body(n_in, prologue, relu_out, *refs):
    xs = [r[...] for r in refs[:n_in]]
    w_ref, b_ref, out_ref = refs[n_in], refs[n_in + 1], refs[n_in + 2]
    if prologue == "none":
        x = xs[0]
    elif prologue == "add":
        x = xs[0] + xs[1]
    elif prologue == "relusub":
        x = jnp.maximum(xs[0] - xs[1] + xs[2], 0.0)
    else:
        raise ValueError(prologue)
    y = jnp.dot(x, w_ref[...], preferred_element_type=jnp.float32)
    if b_ref is not None:
        y = y + b_ref[...]
    if relu_out:
        y = jnp.maximum(y, 0.0)
    out_ref[...] = y


def _mm(xs, w, b=None, prologue="none", relu_out=True, bn=1024):
    n = xs[0].shape[0]
    k = xs[0].shape[1]
    h = w.shape[1]
    nb = pl.cdiv(n, bn)
    n_in = len(xs)
    in_specs = []
    for x in xs:
        if x.shape[0] == 1:
            in_specs.append(pl.BlockSpec((1, x.shape[1]), lambda i: (0, 0)))
        else:
            in_specs.append(pl.BlockSpec((bn, x.shape[1]), lambda i: (i, 0)))
    in_specs.append(pl.BlockSpec((k, h), lambda i: (0, 0)))
    args = list(xs) + [w]
    if b is not None:
        in_specs.append(pl.BlockSpec((1, h), lambda i: (0, 0)))
        args.append(b.reshape(1, h))
    body = functools.partial(_mm_body, n_in, prologue, relu_out)

    def kern(*refs):
        if b is None:
            body(*refs[:n_in + 1], None, refs[-1])
        else:
            body(*refs)

    return pl.pallas_call(
        kern,
        grid=(nb,),
        in_specs=in_specs,
        out_specs=pl.BlockSpec((bn, h), lambda i: (i, 0)),
        out_shape=jax.ShapeDtypeStruct((nb * bn, h), jnp.float32),
    )(*args)[:n]


def _pad_rows(x, bn=1024):
    npad = (-x.shape[0]) % bn
    if npad:
        x = jnp.concatenate([x, jnp.zeros((npad, x.shape[1]), x.dtype)], axis=0)
    return x


def _mm_p(xs, w, b=None, prologue="none", relu_out=True, bn=1024):
    n = xs[0].shape[0]
    xs = [x if x.shape[0] == 1 else _pad_rows(x, bn) for x in xs]
    return _mm(xs, w, b, prologue, relu_out, bn)[:n]


_GC = 128


def _pick_cb(e):
    for c in (4096, 4000, 3200, 2048, 2000, 1600, 1024, 800, 512, 400, 256, 128, 64, 32, 16):
        if e % c == 0:
            return c
    raise ValueError(f"edge count {e} not divisible by any chunk size")


def _pad_cols16(x):
    cpad = (-x.shape[1]) % 16
    if cpad:
        x = jnp.concatenate([x, jnp.zeros((x.shape[0], cpad), x.dtype)], axis=1)
    return x


def _seg_max(v, idx, ci, m, f):
    nv, fp128 = v.shape
    assert fp128 % 128 == 0
    fp = -(-f // 16) * 16
    e = ci.shape[0]
    if idx is None:
        idx = jnp.arange(e, dtype=jnp.int32)
    info = plsc.get_sparse_core_info()
    nw = info.num_cores * info.num_subcores
    ms = -(-m // nw)
    mp = ms * nw
    cb = _pick_cb(e)
    nb = e // cb
    mesh = plsc.VectorSubcoreMesh(core_axis_name="c", subcore_axis_name="s")

    def body(v_hbm, idx_hbm, ci_hbm, out_hbm,
             ci_buf, li_buf, sel_ci, sel_li, rows, acc, sem):
        w = lax.axis_index("s") * info.num_cores + lax.axis_index("c")
        lo = w * ms
        neg = jnp.full((16,), -jnp.inf, jnp.float32)
        iota = lax.iota(jnp.int32, 16)
        zeros16 = jnp.zeros((16,), jnp.int32)
        spill16 = jnp.full((16,), ms, jnp.int32)

        def initb(i, c):
            acc[pl.ds(i * 16, 16)] = neg
            return c
        lax.fori_loop(0, (ms + 1) * fp // 16, initb, jnp.int32(0))

        def chunk(b, c):
            pltpu.sync_copy(ci_hbm.at[pl.ds(b * cb, cb)], ci_buf)
            pltpu.sync_copy(idx_hbm.at[pl.ds(b * cb, cb)], li_buf)

            def scan(g, nsel_v):
                civ = ci_buf[pl.ds(g * 16, 16)]
                liv = li_buf[pl.ds(g * 16, 16)]
                lrow = civ - lo
                msk = (lrow >= 0) & (lrow < ms)
                pos = nsel_v + plsc.cumsum(msk.astype(jnp.int32)) - 1
                plsc.store_scatter(sel_ci, [pos], lrow, mask=msk)
                plsc.store_scatter(sel_li, [pos], liv, mask=msk)
                return nsel_v + plsc.all_reduce_population_count(msk)

            nsel_v = lax.fori_loop(0, cb // 16, scan, zeros16)
            nsel = jnp.max(nsel_v)
            for kz in range(_GC // 16):
                plsc.store_scatter(sel_li, [nsel_v + iota + kz * 16], zeros16)
                plsc.store_scatter(sel_ci, [nsel_v + iota + kz * 16], spill16)
            nt = (nsel + _GC - 1) // _GC

            def gchunk(t, c2):
                pltpu.async_copy(v_hbm.at[sel_li.at[pl.ds(t * _GC, _GC)]], rows, sem).wait()

                def rmw(g, c3):
                    civ = sel_ci[pl.ds(t * _GC + g * 16, 16)]
                    for lane in range(16):
                        base = civ[lane] * fp
                        for cbk in range(fp // 16):
                            aa = acc[pl.ds(base + cbk * 16, 16)]
                            rr = rows[g * 16 + lane, pl.ds(cbk * 16, 16)]
                            acc[pl.ds(base + cbk * 16, 16)] = jnp.maximum(aa, rr)
                    return c3
                lax.fori_loop(0, _GC // 16, rmw, jnp.int32(0))
                return c2
            lax.fori_loop(0, nt, gchunk, jnp.int32(0))
            return c
        lax.fori_loop(0, nb, chunk, jnp.int32(0))
        pltpu.sync_copy(acc.at[pl.ds(0, ms * fp)],
                        out_hbm.at[pl.ds(lo * fp, ms * fp)])

    kf = pl.kernel(
        body,
        out_type=jax.ShapeDtypeStruct((mp * fp,), jnp.float32),
        mesh=mesh,
        compiler_params=pltpu.CompilerParams(needs_layout_passes=False),
        scratch_types=[
            pltpu.VMEM((cb,), jnp.int32),
            pltpu.VMEM((cb,), jnp.int32),
            pltpu.VMEM((cb + _GC,), jnp.int32),
            pltpu.VMEM((cb + _GC,), jnp.int32),
            pltpu.VMEM((_GC, fp128), jnp.float32),
            pltpu.VMEM(((ms + 1) * fp,), jnp.float32),
            pltpu.SemaphoreType.DMA,
        ],
    )
    return kf(v, idx, ci).reshape(mp, fp)[:m]


def _edge_gather_sub(a, c, b1, ci, li):
    f = b1.shape[0]
    fp = -(-f // 16) * 16
    fp128 = a.shape[1]
    assert fp128 % 128 == 0 and c.shape[1] == fp128
    b1p = jnp.pad(b1, (0, fp - f))
    e = ci.shape[0]
    info = plsc.get_sparse_core_info()
    nw = info.num_cores * info.num_subcores
    assert e % nw == 0, e
    epw = e // nw
    gc = max(g for g in range(16, _GC + 1, 16) if epw % g == 0)
    mesh = plsc.VectorSubcoreMesh(core_axis_name="c", subcore_axis_name="s")

    def body(a_hbm, c_hbm, b_hbm, ci_hbm, li_hbm, u_hbm,
             ci_buf, li_buf, b_buf, rows_a, rows_c, u_buf, sem):
        w = lax.axis_index("s") * info.num_cores + lax.axis_index("c")
        base = w * epw
        pltpu.sync_copy(ci_hbm.at[pl.ds(base, epw)], ci_buf)
        pltpu.sync_copy(li_hbm.at[pl.ds(base, epw)], li_buf)
        pltpu.sync_copy(b_hbm, b_buf)

        def gchunk(t, c0):
            pltpu.async_copy(a_hbm.at[li_buf.at[pl.ds(t * gc, gc)]], rows_a, sem).wait()
            pltpu.async_copy(c_hbm.at[ci_buf.at[pl.ds(t * gc, gc)]], rows_c, sem).wait()

            def per_row(q, c1):
                for cbk in range(fp // 16):
                    av = rows_a[q, pl.ds(cbk * 16, 16)]
                    cv = rows_c[q, pl.ds(cbk * 16, 16)]
                    bv = b_buf[pl.ds(cbk * 16, 16)]
                    u_buf[q, pl.ds(cbk * 16, 16)] = jnp.maximum(av - cv + bv, 0.0)
                return c1
            lax.fori_loop(0, gc, per_row, jnp.int32(0))
            pltpu.sync_copy(u_buf, u_hbm.at[pl.ds(base + t * gc, gc)])
            return c0
        lax.fori_loop(0, epw // gc, gchunk, jnp.int32(0))

    kf = pl.kernel(
        body,
        out_type=jax.ShapeDtypeStruct((e, fp), jnp.float32),
        mesh=mesh,
        compiler_params=pltpu.CompilerParams(needs_layout_passes=False),
        scratch_types=[
            pltpu.VMEM((epw,), jnp.int32),
            pltpu.VMEM((epw,), jnp.int32),
            pltpu.VMEM((fp,), jnp.float32),
            pltpu.VMEM((gc, fp128), jnp.float32),
            pltpu.VMEM((gc, fp128), jnp.float32),
            pltpu.VMEM((gc, fp), jnp.float32),
            pltpu.SemaphoreType.DMA,
        ],
    )
    return kf(a, c, b1p.reshape(fp), ci, li)


def _basic_collapsed(in_p, out_p, last_coors, last_features, current_coors,
                     edge, m):
    ci, li = edge[0].astype(jnp.int32), edge[1].astype(jnp.int32)
    f = last_features.shape[1]
    w1, b1 = in_p[0]
    h = w1.shape[1]
    hp128 = -(-h // 128) * 128
    w1p = jnp.pad(w1, ((0, 0), (0, hp128 - h)))
    x_src = jnp.concatenate([last_features, last_coors], axis=1)
    a = _mm_p([x_src], w1p, None, relu_out=False)
    if len(in_p) == 1:
        wc = w1[f:]
        c = _mm_p([current_coors], wc, None, relu_out=False)
        s = _seg_max(a, li, ci, m, h)[:, :h]
        w2, b2 = out_p[0]
        return _mm_p([s, c, b1.reshape(1, -1)], w2, b2, prologue="relusub")
    wc = w1p[f:]
    c = _mm_p([current_coors], wc, None, relu_out=False)
    u = _edge_gather_sub(a, c, b1, ci, li)
    w12, b12 = in_p[1]
    h2 = w12.shape[1]
    h2p128 = -(-h2 // 128) * 128
    w12p = jnp.pad(w12, ((0, u.shape[1] - w12.shape[0]), (0, h2p128 - h2)))
    b12p = jnp.pad(b12, (0, h2p128 - h2))
    h2e = _mm_p([u], w12p, b12p)
    s = _seg_max(h2e, None, ci, m, h2)[:, :h2]
    z = jnp.zeros((1, h2), jnp.float32)
    w2, b2 = out_p[0]
    return _mm_p([s, z, z], w2, b2, prologue="relusub")


def _graph(p, coors, feats, edge):
    upd = _basic_collapsed(p['in'], p['out'], coors, feats, coors, edge,
                           coors.shape[0])
    w, b = p['after'][0]
    return _mm_p([feats, upd], w, b, prologue="add")


def _up(p, cur_c, cur_f, last_c, last_f, edge):
    upd = _basic_collapsed(p['in'], p['out'], cur_c, cur_f, last_c, edge,
                           last_c.shape[0])
    wb, bb = p['before'][0]
    before = _mm_p([last_f], wb, bb)
    wa, ba = p['after'][0]
    return _mm_p([before, upd], wa, ba, prologue="add")


def kernel(points, coors1, coors2, coors3, params, edge_0_1, edge_1_1,
           edge_1_2, edge_2_2, edge_2_3, edge_3_3, edge_3_2, edge_2_1,
           edge_1_0):
    p = params
    c0 = points[:, :3]
    n1, n2, n3 = coors1.shape[0], coors2.shape[0], coors3.shape[0]
    f1 = _basic_collapsed(p['downsample1']['in'], p['downsample1']['out'],
                          c0, points, coors1, edge_0_1, n1)
    f1 = _graph(p['graph1'], coors1, f1, edge_1_1)
    f2 = _basic_collapsed(p['downsample2']['in'], p['downsample2']['out'],
                          coors1, f1, coors2, edge_1_2, n2)
    f2 = _graph(p['graph2'], coors2, f2, edge_2_2)
    f3 = _basic_collapsed(p['downsample3']['in'], p['downsample3']['out'],
                          coors2, f2, coors3, edge_2_3, n3)
    f3 = _graph(p['graph3'], coors3, f3, edge_3_3)
    f2u = _up(p['upsample1'], coors3, f3, coors2, f2, edge_3_2)
    f2u = _graph(p['graph2_update'], coors2, f2u, edge_2_2)
    f1u = _up(p['upsample2'], coors2, f2u, coors1, f1, edge_2_1)
    f1u = _graph(p['graph1_update'], coors1, f1u, edge_1_1)
    return _up(p['upsample3'], coors1, f1u, c0, points, edge_1_0)


# speedup vs baseline: 2.0867x; 2.0867x over previous
import functools

import jax
import jax.numpy as jnp
from jax import lax
from jax.experimental import pallas as pl
from jax.experimental.pallas import tpu as pltpu
from jax.experimental.pallas import tpu_sc as plsc


def _mm_body(n_in, prologue, relu_out, *refs):
    xs = [r[...] for r in refs[:n_in]]
    w_ref, b_ref, out_ref = refs[n_in], refs[n_in + 1], refs[n_in + 2]
    if prologue == "none":
        x = xs[0]
    elif prologue == "add":
        x = xs[0] + xs[1]
    elif prologue == "relusub":
        x = jnp.maximum(xs[0] - xs[1] + xs[2], 0.0)
    else:
        raise ValueError(prologue)
    y = jnp.dot(x, w_ref[...], preferred_element_type=jnp.float32)
    if b_ref is not None:
        y = y + b_ref[...]
    if relu_out:
        y = jnp.maximum(y, 0.0)
    out_ref[...] = y


def _mm(xs, w, b=None, prologue="none", relu_out=True, bn=1024):
    n = xs[0].shape[0]
    k = xs[0].shape[1]
    h = w.shape[1]
    nb = pl.cdiv(n, bn)
    n_in = len(xs)
    in_specs = []
    for x in xs:
        if x.shape[0] == 1:
            in_specs.append(pl.BlockSpec((1, x.shape[1]), lambda i: (0, 0)))
        else:
            in_specs.append(pl.BlockSpec((bn, x.shape[1]), lambda i: (i, 0)))
    in_specs.append(pl.BlockSpec((k, h), lambda i: (0, 0)))
    args = list(xs) + [w]
    if b is not None:
        in_specs.append(pl.BlockSpec((1, h), lambda i: (0, 0)))
        args.append(b.reshape(1, h))
    body = functools.partial(_mm_body, n_in, prologue, relu_out)

    def kern(*refs):
        if b is None:
            body(*refs[:n_in + 1], None, refs[-1])
        else:
            body(*refs)

    return pl.pallas_call(
        kern,
        grid=(nb,),
        in_specs=in_specs,
        out_specs=pl.BlockSpec((bn, h), lambda i: (i, 0)),
        out_shape=jax.ShapeDtypeStruct((nb * bn, h), jnp.float32),
    )(*args)[:n]


def _pad_rows(x, bn=1024):
    npad = (-x.shape[0]) % bn
    if npad:
        x = jnp.concatenate([x, jnp.zeros((npad, x.shape[1]), x.dtype)], axis=0)
    return x


def _mm_p(xs, w, b=None, prologue="none", relu_out=True, bn=1024):
    n = xs[0].shape[0]
    xs = [x if x.shape[0] == 1 else _pad_rows(x, bn) for x in xs]
    return _mm(xs, w, b, prologue, relu_out, bn)[:n]


_GC = 128


def _pick_cb(e, budget_words):
    for c in (16000, 8000, 4096, 4000, 3200, 2048, 2000, 1600, 1024, 800, 512,
              400, 256, 128, 64, 32, 16):
        if e % c == 0 and 4 * c + 2 * _GC <= budget_words:
            return c
    raise ValueError(f"edge count {e} not divisible by any chunk size")


def _pad_cols16(x):
    cpad = (-x.shape[1]) % 16
    if cpad:
        x = jnp.concatenate([x, jnp.zeros((x.shape[0], cpad), x.dtype)], axis=1)
    return x


def _seg_max(v, idx, ci, m, f):
    nv, fp128 = v.shape
    assert fp128 % 128 == 0
    fp = -(-f // 16) * 16
    e = ci.shape[0]
    if idx is None:
        idx = jnp.arange(e, dtype=jnp.int32)
    info = plsc.get_sparse_core_info()
    nw = info.num_cores * info.num_subcores
    ms = -(-m // nw)
    mp = ms * nw
    budget = 120000 - _GC * fp128 - (ms + 1) * fp
    cb = _pick_cb(e, budget)
    nb = e // cb
    mesh = plsc.VectorSubcoreMesh(core_axis_name="c", subcore_axis_name="s")

    def body(v_hbm, idx_hbm, ci_hbm, out_hbm,
             ci_buf, li_buf, sel_ci, sel_li, rows, acc, sem):
        w = lax.axis_index("s") * info.num_cores + lax.axis_index("c")
        lo = w * ms
        neg = jnp.full((16,), -jnp.inf, jnp.float32)
        iota = lax.iota(jnp.int32, 16)
        zeros16 = jnp.zeros((16,), jnp.int32)
        spill16 = jnp.full((16,), ms, jnp.int32)

        def initb(i, c):
            acc[pl.ds(i * 16, 16)] = neg
            return c
        lax.fori_loop(0, (ms + 1) * fp // 16, initb, jnp.int32(0))

        def chunk(b, c):
            pltpu.sync_copy(ci_hbm.at[pl.ds(b * cb, cb)], ci_buf)
            pltpu.sync_copy(idx_hbm.at[pl.ds(b * cb, cb)], li_buf)

            def scan(g, nsel):
                civ = ci_buf[pl.ds(g * 16, 16)]
                liv = li_buf[pl.ds(g * 16, 16)]
                lrow = civ - lo
                msk = (lrow >= 0) & (lrow < ms)
                plsc.store_compressed(sel_ci.at[pl.ds(nsel, 16)], lrow, mask=msk)
                plsc.store_compressed(sel_li.at[pl.ds(nsel, 16)], liv, mask=msk)
                return nsel + plsc.all_reduce_population_count(msk)[0]

            nsel = lax.fori_loop(0, cb // 16, scan, jnp.int32(0))
            pos0 = jnp.broadcast_to(nsel, (16,)).astype(jnp.int32) + iota
            for kz in range(_GC // 16):
                plsc.store_scatter(sel_li, [pos0 + kz * 16], zeros16)
                plsc.store_scatter(sel_ci, [pos0 + kz * 16], spill16)
            nt = (nsel + _GC - 1) // _GC

            def gchunk(t, c2):
                pltpu.async_copy(v_hbm.at[sel_li.at[pl.ds(t * _GC, _GC)]], rows, sem).wait()

                def rmw(g, c3):
                    civ = sel_ci[pl.ds(t * _GC + g * 16, 16)]
                    for lane in range(16):
                        base = civ[lane] * fp
                        for cbk in range(fp // 16):
                            aa = acc[pl.ds(base + cbk * 16, 16)]
                            rr = rows[g * 16 + lane, pl.ds(cbk * 16, 16)]
                            acc[pl.ds(base + cbk * 16, 16)] = jnp.maximum(aa, rr)
                    return c3
                lax.fori_loop(0, _GC // 16, rmw, jnp.int32(0))
                return c2
            lax.fori_loop(0, nt, gchunk, jnp.int32(0))
            return c
        lax.fori_loop(0, nb, chunk, jnp.int32(0))
        pltpu.sync_copy(acc.at[pl.ds(0, ms * fp)],
                        out_hbm.at[pl.ds(lo * fp, ms * fp)])

    kf = pl.kernel(
        body,
        out_type=jax.ShapeDtypeStruct((mp * fp,), jnp.float32),
        mesh=mesh,
        compiler_params=pltpu.CompilerParams(needs_layout_passes=False),
        scratch_types=[
            pltpu.VMEM((cb,), jnp.int32),
            pltpu.VMEM((cb,), jnp.int32),
            pltpu.VMEM((cb + _GC,), jnp.int32),
            pltpu.VMEM((cb + _GC,), jnp.int32),
            pltpu.VMEM((_GC, fp128), jnp.float32),
            pltpu.VMEM(((ms + 1) * fp,), jnp.float32),
            pltpu.SemaphoreType.DMA,
        ],
    )
    return kf(v, idx, ci).reshape(mp, fp)[:m]


def _edge_gather_sub(a, c, b1, ci, li):
    f = b1.shape[0]
    fp = -(-f // 16) * 16
    fp128 = a.shape[1]
    assert fp128 % 128 == 0 and c.shape[1] == fp128
    b1p = jnp.pad(b1, (0, fp - f))
    e = ci.shape[0]
    info = plsc.get_sparse_core_info()
    nw = info.num_cores * info.num_subcores
    assert e % nw == 0, e
    epw = e // nw
    gc = max(g for g in range(16, _GC + 1, 16) if epw % g == 0)
    mesh = plsc.VectorSubcoreMesh(core_axis_name="c", subcore_axis_name="s")

    def body(a_hbm, c_hbm, b_hbm, ci_hbm, li_hbm, u_hbm,
             ci_buf, li_buf, b_buf, rows_a, rows_c, u_buf, sem):
        w = lax.axis_index("s") * info.num_cores + lax.axis_index("c")
        base = w * epw
        pltpu.sync_copy(ci_hbm.at[pl.ds(base, epw)], ci_buf)
        pltpu.sync_copy(li_hbm.at[pl.ds(base, epw)], li_buf)
        pltpu.sync_copy(b_hbm, b_buf)

        def gchunk(t, c0):
            pltpu.async_copy(a_hbm.at[li_buf.at[pl.ds(t * gc, gc)]], rows_a, sem).wait()
            pltpu.async_copy(c_hbm.at[ci_buf.at[pl.ds(t * gc, gc)]], rows_c, sem).wait()

            def per_row(q, c1):
                for cbk in range(fp // 16):
                    av = rows_a[q, pl.ds(cbk * 16, 16)]
                    cv = rows_c[q, pl.ds(cbk * 16, 16)]
                    bv = b_buf[pl.ds(cbk * 16, 16)]
                    u_buf[q, pl.ds(cbk * 16, 16)] = jnp.maximum(av - cv + bv, 0.0)
                return c1
            lax.fori_loop(0, gc, per_row, jnp.int32(0))
            pltpu.sync_copy(u_buf, u_hbm.at[pl.ds(base + t * gc, gc)])
            return c0
        lax.fori_loop(0, epw // gc, gchunk, jnp.int32(0))

    kf = pl.kernel(
        body,
        out_type=jax.ShapeDtypeStruct((e, fp), jnp.float32),
        mesh=mesh,
        compiler_params=pltpu.CompilerParams(needs_layout_passes=False),
        scratch_types=[
            pltpu.VMEM((epw,), jnp.int32),
            pltpu.VMEM((epw,), jnp.int32),
            pltpu.VMEM((fp,), jnp.float32),
            pltpu.VMEM((gc, fp128), jnp.float32),
            pltpu.VMEM((gc, fp128), jnp.float32),
            pltpu.VMEM((gc, fp), jnp.float32),
            pltpu.SemaphoreType.DMA,
        ],
    )
    return kf(a, c, b1p.reshape(fp), ci, li)


def _basic_collapsed(in_p, out_p, last_coors, last_features, current_coors,
                     edge, m):
    ci, li = edge[0].astype(jnp.int32), edge[1].astype(jnp.int32)
    f = last_features.shape[1]
    w1, b1 = in_p[0]
    h = w1.shape[1]
    hp128 = -(-h // 128) * 128
    w1p = jnp.pad(w1, ((0, 0), (0, hp128 - h)))
    x_src = jnp.concatenate([last_features, last_coors], axis=1)
    a = _mm_p([x_src], w1p, None, relu_out=False)
    if len(in_p) == 1:
        wc = w1[f:]
        c = _mm_p([current_coors], wc, None, relu_out=False)
        s = _seg_max(a, li, ci, m, h)[:, :h]
        w2, b2 = out_p[0]
        return _mm_p([s, c, b1.reshape(1, -1)], w2, b2, prologue="relusub")
    wc = w1p[f:]
    c = _mm_p([current_coors], wc, None, relu_out=False)
    u = _edge_gather_sub(a, c, b1, ci, li)
    w12, b12 = in_p[1]
    h2 = w12.shape[1]
    h2p128 = -(-h2 // 128) * 128
    w12p = jnp.pad(w12, ((0, u.shape[1] - w12.shape[0]), (0, h2p128 - h2)))
    b12p = jnp.pad(b12, (0, h2p128 - h2))
    h2e = _mm_p([u], w12p, b12p)
    s = _seg_max(h2e, None, ci, m, h2)[:, :h2]
    z = jnp.zeros((1, h2), jnp.float32)
    w2, b2 = out_p[0]
    return _mm_p([s, z, z], w2, b2, prologue="relusub")


def _graph(p, coors, feats, edge):
    upd = _basic_collapsed(p['in'], p['out'], coors, feats, coors, edge,
                           coors.shape[0])
    w, b = p['after'][0]
    return _mm_p([feats, upd], w, b, prologue="add")


def _up(p, cur_c, cur_f, last_c, last_f, edge):
    upd = _basic_collapsed(p['in'], p['out'], cur_c, cur_f, last_c, edge,
                           last_c.shape[0])
    wb, bb = p['before'][0]
    before = _mm_p([last_f], wb, bb)
    wa, ba = p['after'][0]
    return _mm_p([before, upd], wa, ba, prologue="add")


def kernel(points, coors1, coors2, coors3, params, edge_0_1, edge_1_1,
           edge_1_2, edge_2_2, edge_2_3, edge_3_3, edge_3_2, edge_2_1,
           edge_1_0):
    p = params
    c0 = points[:, :3]
    n1, n2, n3 = coors1.shape[0], coors2.shape[0], coors3.shape[0]
    f1 = _basic_collapsed(p['downsample1']['in'], p['downsample1']['out'],
                          c0, points, coors1, edge_0_1, n1)
    f1 = _graph(p['graph1'], coors1, f1, edge_1_1)
    f2 = _basic_collapsed(p['downsample2']['in'], p['downsample2']['out'],
                          coors1, f1, coors2, edge_1_2, n2)
    f2 = _graph(p['graph2'], coors2, f2, edge_2_2)
    f3 = _basic_collapsed(p['downsample3']['in'], p['downsample3']['out'],
                          coors2, f2, coors3, edge_2_3, n3)
    f3 = _graph(p['graph3'], coors3, f3, edge_3_3)
    f2u = _up(p['upsample1'], coors3, f3, coors2, f2, edge_3_2)
    f2u = _graph(p['graph2_update'], coors2, f2u, edge_2_2)
    f1u = _up(p['upsample2'], coors2, f2u, coors1, f1, edge_2_1)
    f1u = _graph(p['graph1_update'], coors1, f1u, edge_1_1)
    return _up(p['upsample3'], coors1, f1u, c0, points, edge_1_0)
